# baseline (speedup 1.0000x reference)
import functools
import jax
import jax.numpy as jnp
from jax import lax
from jax.experimental import pallas as pl
from jax.experimental.pallas import tpu as pltpu
from jax.experimental.pallas import tpu_sc as plsc

N_USERS = 100000
N_ITEMS = 100000
K = 64
B = 16384

_info = plsc.get_sparse_core_info()
_NC, _NS, _L = _info.num_cores, _info.num_subcores, _info.num_lanes
_NW = _NC * _NS
_BPW = B // _NW


def _mf_kernel(users_hbm, items_hbm, U_hbm, M_hbm, Ub_hbm, Mb_hbm, out_hbm,
               idx_u, idx_i, u_rows, m_rows, ub_v, mb_v, out_v,
               sem_u, sem_m, sem_ub, sem_mb):
    wid = lax.axis_index("s") * _NC + lax.axis_index("c")
    base = wid * _BPW

    pltpu.sync_copy(users_hbm.at[pl.ds(base, _BPW)], idx_u)
    pltpu.sync_copy(items_hbm.at[pl.ds(base, _BPW)], idx_i)

    cp_u = pltpu.async_copy(U_hbm.at[idx_u], u_rows, sem_u)
    cp_m = pltpu.async_copy(M_hbm.at[idx_i], m_rows, sem_m)
    cp_ub = pltpu.async_copy(Ub_hbm.at[idx_u], ub_v, sem_ub)
    cp_mb = pltpu.async_copy(Mb_hbm.at[idx_i], mb_v, sem_mb)
    cp_u.wait()
    cp_m.wait()
    cp_ub.wait()
    cp_mb.wait()

    lane = lax.iota(jnp.int32, _L)

    def group_body(g, _):
        gb = g * _L
        res = jnp.zeros((_L,), jnp.float32)
        for r in range(_L):
            j = gb + r
            acc = None
            for t in range(K // _L):
                u = u_rows[j, pl.ds(t * _L, _L)]
                m = m_rows[j, pl.ds(t * _L, _L)]
                p = u * m
                acc = p if acc is None else acc + p
            s = jnp.sum(acc)
            res = jnp.where(lane == r, s, res)
        res = res + ub_v[pl.ds(gb, _L)] + mb_v[pl.ds(gb, _L)]
        out_v[pl.ds(gb, _L)] = res
        return 0

    lax.fori_loop(0, _BPW // _L, group_body, 0)

    pltpu.sync_copy(out_v, out_hbm.at[pl.ds(base, _BPW)])


@jax.jit
def _run(users, items, U, M, Ub, Mb):
    mesh = plsc.VectorSubcoreMesh(core_axis_name="c", subcore_axis_name="s")
    kfn = functools.partial(
        pl.kernel,
        out_type=jax.ShapeDtypeStruct((B,), jnp.float32),
        mesh=mesh,
        scratch_types=[
            pltpu.VMEM((_BPW,), jnp.int32),
            pltpu.VMEM((_BPW,), jnp.int32),
            pltpu.VMEM((_BPW, K), jnp.float32),
            pltpu.VMEM((_BPW, K), jnp.float32),
            pltpu.VMEM((_BPW,), jnp.float32),
            pltpu.VMEM((_BPW,), jnp.float32),
            pltpu.VMEM((_BPW,), jnp.float32),
            pltpu.SemaphoreType.DMA,
            pltpu.SemaphoreType.DMA,
            pltpu.SemaphoreType.DMA,
            pltpu.SemaphoreType.DMA,
        ],
        compiler_params=pltpu.CompilerParams(
            needs_layout_passes=False, use_tc_tiling_on_sc=False),
    )(_mf_kernel)
    return kfn(users, items, U, M, Ub, Mb)


def kernel(users, items, movie_map, U, M, Ub, Mb):
    del movie_map
    return _run(users.astype(jnp.int32), items.astype(jnp.int32),
                U, M, Ub.reshape(-1), Mb.reshape(-1))


# speedup vs baseline: 1.2291x; 1.2291x over previous
import functools
import jax
import jax.numpy as jnp
from jax import lax
from jax.experimental import pallas as pl
from jax.experimental.pallas import tpu as pltpu
from jax.experimental.pallas import tpu_sc as plsc

N_USERS = 100000
N_ITEMS = 100000
K = 64
B = 16384

_info = plsc.get_sparse_core_info()
_NC, _NS, _L = _info.num_cores, _info.num_subcores, _info.num_lanes
_NW = _NC * _NS
_BPW = B // _NW
_CH = 256
_NCH = _BPW // _CH
_NG = _CH // _L


def _mf_kernel(users_hbm, items_hbm, U_hbm, M_hbm, Ub_hbm, Mb_hbm, out_hbm,
               idx_uv, idx_iv, u_rows, m_rows, ub_v, mb_v,
               out_v, sem_u, sem_m, sem_b):
    wid = lax.axis_index("s") * _NC + lax.axis_index("c")
    base = wid * _BPW

    pltpu.sync_copy(users_hbm.at[pl.ds(base, _BPW)], idx_uv)
    pltpu.sync_copy(items_hbm.at[pl.ds(base, _BPW)], idx_iv)

    cp_ub = pltpu.async_copy(Ub_hbm.at[idx_uv], ub_v, sem_b)
    cp_mb = pltpu.async_copy(Mb_hbm.at[idx_iv], mb_v, sem_b)

    lane = lax.iota(jnp.int32, _L)

    def chunk_body(c, _):
        cb = c * _CH

        def fire(g):
            gb = g * _L
            ru = idx_uv[pl.ds(cb + gb, _L)]
            ri = idx_iv[pl.ds(cb + gb, _L)]
            for r in range(_L):
                pltpu.async_copy(U_hbm.at[pl.ds(ru[r], 1), :],
                                 u_rows.at[pl.ds(gb + r, 1), :], sem_u)
                pltpu.async_copy(M_hbm.at[pl.ds(ri[r], 1), :],
                                 m_rows.at[pl.ds(gb + r, 1), :], sem_m)

        def drain(g):
            gb = g * _L
            for r in range(_L):
                pltpu.make_async_copy(
                    U_hbm.at[pl.ds(0, 1), :],
                    u_rows.at[pl.ds(gb + r, 1), :], sem_u).wait()
                pltpu.make_async_copy(
                    M_hbm.at[pl.ds(0, 1), :],
                    m_rows.at[pl.ds(gb + r, 1), :], sem_m).wait()

        fire(0)

        def dma_body(g, _):
            fire(g)
            drain(g - 1)
            return 0

        lax.fori_loop(1, _NG, dma_body, 0)
        drain(_NG - 1)

        def group_body(g, _):
            gb = g * _L
            res = jnp.zeros((_L,), jnp.float32)
            for r in range(_L):
                j = gb + r
                acc = None
                for t in range(K // _L):
                    u = u_rows[j, pl.ds(t * _L, _L)]
                    m = m_rows[j, pl.ds(t * _L, _L)]
                    p = u * m
                    acc = p if acc is None else acc + p
                s = jnp.sum(acc)
                res = jnp.where(lane == r, s, res)
            res = res + ub_v[pl.ds(cb + gb, _L)] + mb_v[pl.ds(cb + gb, _L)]
            out_v[pl.ds(cb + gb, _L)] = res
            return 0

        lax.fori_loop(0, _NG, group_body, 0)
        return 0

    cp_ub.wait()
    cp_mb.wait()
    lax.fori_loop(0, _NCH, chunk_body, 0)

    pltpu.sync_copy(out_v, out_hbm.at[pl.ds(base, _BPW)])


@jax.jit
def _run(users, items, U, M, Ub, Mb):
    mesh = plsc.VectorSubcoreMesh(core_axis_name="c", subcore_axis_name="s")
    kfn = functools.partial(
        pl.kernel,
        out_type=jax.ShapeDtypeStruct((B,), jnp.float32),
        mesh=mesh,
        scratch_types=[
            pltpu.VMEM((_BPW,), jnp.int32),
            pltpu.VMEM((_BPW,), jnp.int32),
            pltpu.VMEM((_CH, K), jnp.float32),
            pltpu.VMEM((_CH, K), jnp.float32),
            pltpu.VMEM((_BPW,), jnp.float32),
            pltpu.VMEM((_BPW,), jnp.float32),
            pltpu.VMEM((_BPW,), jnp.float32),
            pltpu.SemaphoreType.DMA,
            pltpu.SemaphoreType.DMA,
            pltpu.SemaphoreType.DMA,
        ],
        compiler_params=pltpu.CompilerParams(needs_layout_passes=False),
    )(_mf_kernel)
    return kfn(users, items, U, M, Ub, Mb)


def kernel(users, items, movie_map, U, M, Ub, Mb):
    del movie_map
    return _run(users.astype(jnp.int32), items.astype(jnp.int32),
                U, M, Ub.reshape(-1), Mb.reshape(-1))
